# baseline (speedup 1.0000x reference)
import functools

import jax
import jax.numpy as jnp
from jax import lax
from jax.experimental import pallas as pl
from jax.experimental.pallas import tpu as pltpu
from jax.experimental.pallas import tpu_sc as plsc

B = 16384
D = 64
NS = 16
H = 62500
NG = B // 16
UNROLL = 4
CAP = B + 16


_mesh = plsc.VectorSubcoreMesh(
    core_axis_name="c", subcore_axis_name="s", num_cores=1)


@functools.partial(
    pl.kernel,
    mesh=_mesh,
    out_type=jax.ShapeDtypeStruct((B, D), jnp.float32),
    compiler_params=pltpu.CompilerParams(
        needs_layout_passes=False, use_tc_tiling_on_sc=False),
    scratch_types=[
        pltpu.VMEM((B,), jnp.int32),
        pltpu.VMEM((CAP,), jnp.int32),
        pltpu.VMEM((H,), jnp.int32),
        pltpu.VMEM((128,), jnp.int32),
        pltpu.VMEM((128,), jnp.int32),
        pltpu.VMEM((128, D), jnp.float32),
        pltpu.SemaphoreType.DMA,
    ],
)
def _sc_update_gather(ids_hbm, msgs_hbm, out_hbm,
                      ids_v, occ_v, tloc_v, widx_v, oidx_v, rows_v, sem):
    s = lax.axis_index("s")
    lane = lax.iota(jnp.int32, 16)

    pltpu.sync_copy(ids_hbm, ids_v)

    def scan_step(gg, ptr):
        for u in range(UNROLL):
            g = gg * UNROLL + u
            ids = ids_v[pl.ds(g * 16, 16)]
            own = (ids & 15) == s
            v = ((ids >> 4) << 14) + (g * 16 + lane)
            plsc.store_compressed(occ_v.at[pl.ds(ptr, 16)], v, mask=own)
            ptr = ptr + jnp.sum(own.astype(jnp.int32))
        return ptr

    n = lax.fori_loop(0, NG // UNROLL, scan_step, jnp.int32(0))

    shift_idx = [jnp.minimum(lane + k, 15) for k in range(1, 16)]
    shift_ok = [lane + k <= 15 for k in range(1, 16)]

    def table_step(g, carry):
        nv = n - g * 16
        valid = lane < nv
        v = jnp.where(valid, occ_v[pl.ds(g * 16, 16)], 0)
        h = v >> 14
        beaten = h != h
        for k in range(15):
            nb = h.at[shift_idx[k]].get(mode="promise_in_bounds")
            beaten = beaten | ((nb == h) & shift_ok[k] & ((lane + k + 1) < nv))
        keep = (~beaten) & valid
        plsc.store_scatter(tloc_v, [h], v & 16383, mask=keep)
        return carry

    lax.fori_loop(0, (n + 15) // 16, table_step, 0)

    zeros16 = lane * 0

    def emit_step(j, carry):
        w0 = zeros16
        o0 = zeros16
        for k in range(8):
            off = j * 128 + k * 16
            valid = (off + lane) < n
            v = jnp.where(valid, occ_v[pl.ds(off, 16)], 0)
            h = v >> 14
            i = v & 16383
            w = plsc.load_gather(tloc_v, [h])
            if k == 0:
                w0 = w.at[zeros16].get(mode="promise_in_bounds")
                o0 = i.at[zeros16].get(mode="promise_in_bounds")
            widx_v[pl.ds(k * 16, 16)] = jnp.where(valid, w, w0)
            oidx_v[pl.ds(k * 16, 16)] = jnp.where(valid, i, o0)
        pltpu.async_copy(msgs_hbm.at[widx_v], rows_v, sem).wait()
        pltpu.async_copy(rows_v, out_hbm.at[oidx_v], sem).wait()
        return carry

    lax.fori_loop(0, (n + 127) // 128, emit_step, 0)


def kernel(node_ids, messages, timestamps, memory, last_update):
    del timestamps, memory, last_update
    return _sc_update_gather(node_ids, messages)
